# Initial kernel scaffold; baseline (speedup 1.0000x reference)
import jax
import jax.numpy as jnp
from jax.experimental import pallas as pl


def kernel(token, W):
    raise NotImplementedError("write your pallas kernel here")


# speedup vs baseline: 1.4033x; 1.4033x over previous
import functools

import jax
import jax.numpy as jnp
from jax import lax
from jax.experimental import pallas as pl
from jax.experimental.pallas import tpu as pltpu
from jax.experimental.pallas import tpu_sc as plsc

N_EMBD = 128
HALF = N_EMBD // 2
L = 16
NC = 2
NS = 16
NW = NC * NS
IDX_CHUNK = 128


def _rotary_tables(seq_len):
    inv_freq = 1.0 / (10000.0 ** (jnp.arange(0, N_EMBD, 2, dtype=jnp.float32) / N_EMBD))
    seq = jnp.arange(seq_len, dtype=jnp.float32)
    freqs = seq[:, None] * inv_freq[None, :]
    pos = jnp.concatenate((freqs, freqs), axis=-1)
    return jnp.cos(pos), jnp.sin(pos)


def _make_sc_kernel(batch, seq_len):
    total = batch * seq_len
    b_per_w = total // NW
    n_gather = b_per_w // IDX_CHUNK

    mesh = plsc.VectorSubcoreMesh(
        core_axis_name="c", subcore_axis_name="s", num_cores=NC, num_subcores=NS
    )

    @functools.partial(
        pl.kernel,
        out_type=jax.ShapeDtypeStruct((total, N_EMBD), jnp.float32),
        mesh=mesh,
        scratch_types=[
            pltpu.VMEM((n_gather, IDX_CHUNK), jnp.int32),
            pltpu.VMEM((b_per_w, N_EMBD), jnp.float32),
            pltpu.VMEM((b_per_w, N_EMBD), jnp.float32),
            pltpu.VMEM((b_per_w, N_EMBD), jnp.float32),
            pltpu.SemaphoreType.DMA,
        ],
    )
    def sc_kernel(tok_hbm, w_hbm, cos_hbm, sin_hbm, out_hbm,
                  idx_v, rows_v, cos_v, sin_v, sem):
        wid = lax.axis_index("s") * NC + lax.axis_index("c")
        base = wid * b_per_w
        pos_base = lax.rem(base, seq_len)

        pltpu.sync_copy(tok_hbm.at[pl.ds(wid * n_gather, n_gather)], idx_v)
        pltpu.sync_copy(cos_hbm.at[pl.ds(pos_base, b_per_w)], cos_v)
        pltpu.sync_copy(sin_hbm.at[pl.ds(pos_base, b_per_w)], sin_v)

        copies = [
            pltpu.async_copy(
                w_hbm.at[idx_v.at[g]],
                rows_v.at[pl.ds(g * IDX_CHUNK, IDX_CHUNK)],
                sem,
            )
            for g in range(n_gather)
        ]
        for c in copies:
            c.wait()

        def body(t, carry):
            ts = [rows_v[t, pl.ds(j * L, L)] for j in range(N_EMBD // L)]
            cs = [cos_v[t, pl.ds(j * L, L)] for j in range(N_EMBD // L)]
            ss = [sin_v[t, pl.ds(j * L, L)] for j in range(N_EMBD // L)]
            half = HALF // L
            for j in range(N_EMBD // L):
                if j < half:
                    o = ts[j] * cs[j] - ts[j + half] * ss[j]
                else:
                    o = ts[j] * cs[j] + ts[j - half] * ss[j]
                rows_v[t, pl.ds(j * L, L)] = o
            return carry

        lax.fori_loop(0, b_per_w, body, 0)

        pltpu.sync_copy(rows_v, out_hbm.at[pl.ds(base, b_per_w)])

    return sc_kernel


def kernel(token, W):
    batch, seq_len = token.shape
    cos, sin = _rotary_tables(seq_len)
    sc = _make_sc_kernel(batch, seq_len)
    out = sc(token.reshape(-1, IDX_CHUNK), W, cos, sin)
    return out.reshape(batch, seq_len, N_EMBD)


# speedup vs baseline: 1.5840x; 1.1287x over previous
import functools

import jax
import jax.numpy as jnp
import numpy as np
from jax import lax
from jax.experimental import pallas as pl
from jax.experimental.pallas import tpu as pltpu
from jax.experimental.pallas import tpu_sc as plsc

N_EMBD = 128
HALF = N_EMBD // 2
L = 16
NC = 2
NS = 16
NW = NC * NS
IDX_CHUNK = 128


def _rotary_half_tables(seq_len):
    inv_freq = 1.0 / (10000.0 ** (np.arange(0, N_EMBD, 2, dtype=np.float32) / N_EMBD))
    freqs = np.arange(seq_len, dtype=np.float32)[:, None] * inv_freq[None, :]
    return jnp.asarray(np.cos(freqs)), jnp.asarray(np.sin(freqs))


def _make_sc_kernel(batch, seq_len):
    total = batch * seq_len
    b_per_w = total // NW
    n_gather = b_per_w // IDX_CHUNK

    mesh = plsc.VectorSubcoreMesh(
        core_axis_name="c", subcore_axis_name="s", num_cores=NC, num_subcores=NS
    )

    @functools.partial(
        pl.kernel,
        out_type=jax.ShapeDtypeStruct((total, N_EMBD), jnp.float32),
        mesh=mesh,
        scratch_types=[
            pltpu.VMEM((n_gather, IDX_CHUNK), jnp.int32),
            pltpu.VMEM((b_per_w, N_EMBD), jnp.float32),
            pltpu.VMEM((b_per_w, HALF), jnp.float32),
            pltpu.VMEM((b_per_w, HALF), jnp.float32),
            pltpu.SemaphoreType.DMA,
            pltpu.SemaphoreType.DMA,
            pltpu.SemaphoreType.DMA,
            pltpu.SemaphoreType.DMA,
        ],
    )
    def sc_kernel(tok_hbm, w_hbm, cos_hbm, sin_hbm, out_hbm,
                  idx_v, rows_v, cos_v, sin_v, sem_t, sem_g0, sem_g1, sem_w):
        wid = lax.axis_index("s") * NC + lax.axis_index("c")
        base = wid * b_per_w
        pos_base = lax.rem(base, seq_len)

        pltpu.sync_copy(tok_hbm.at[pl.ds(wid * n_gather, n_gather)], idx_v)

        tab = [
            pltpu.async_copy(cos_hbm.at[pl.ds(pos_base, b_per_w)], cos_v, sem_t),
            pltpu.async_copy(sin_hbm.at[pl.ds(pos_base, b_per_w)], sin_v, sem_t),
        ]
        gsems = [sem_g0, sem_g1]
        gathers = [
            pltpu.async_copy(
                w_hbm.at[idx_v.at[g]],
                rows_v.at[pl.ds(g * IDX_CHUNK, IDX_CHUNK)],
                gsems[g],
            )
            for g in range(n_gather)
        ]
        for t in tab:
            t.wait()

        def rotate(t, carry):
            ts = [rows_v[t, pl.ds(j * L, L)] for j in range(N_EMBD // L)]
            cs = [cos_v[t, pl.ds(j * L, L)] for j in range(HALF // L)]
            ss = [sin_v[t, pl.ds(j * L, L)] for j in range(HALF // L)]
            half = HALF // L
            for j in range(half):
                rows_v[t, pl.ds(j * L, L)] = ts[j] * cs[j] - ts[j + half] * ss[j]
                rows_v[t, pl.ds((j + half) * L, L)] = (
                    ts[j + half] * cs[j] + ts[j] * ss[j]
                )
            return carry

        writes = []
        for g in range(n_gather):
            gathers[g].wait()
            lax.fori_loop(g * IDX_CHUNK, (g + 1) * IDX_CHUNK, rotate, 0)
            writes.append(
                pltpu.async_copy(
                    rows_v.at[pl.ds(g * IDX_CHUNK, IDX_CHUNK)],
                    out_hbm.at[pl.ds(base + g * IDX_CHUNK, IDX_CHUNK)],
                    sem_w,
                )
            )
        for w in writes:
            w.wait()

    return sc_kernel


def kernel(token, W):
    batch, seq_len = token.shape
    cos, sin = _rotary_half_tables(seq_len)
    sc = _make_sc_kernel(batch, seq_len)
    out = sc(token.reshape(-1, IDX_CHUNK), W, cos, sin)
    return out.reshape(batch, seq_len, N_EMBD)


# speedup vs baseline: 1.8627x; 1.1760x over previous
import functools

import jax
import jax.numpy as jnp
import numpy as np
from jax import lax
from jax.experimental import pallas as pl
from jax.experimental.pallas import tpu as pltpu
from jax.experimental.pallas import tpu_sc as plsc

N_EMBD = 128
HALF = N_EMBD // 2
L = 16
NC = 2
NS = 16
NW = NC * NS
IDX_CHUNK = 128


def _rotary_cs_table(seq_len):
    inv_freq = 1.0 / (10000.0 ** (np.arange(0, N_EMBD, 2, dtype=np.float32) / N_EMBD))
    freqs = np.arange(seq_len, dtype=np.float32)[:, None] * inv_freq[None, :]
    return jnp.asarray(np.concatenate([np.cos(freqs), np.sin(freqs)], axis=1))


def _make_sc_kernel(batch, seq_len):
    total = batch * seq_len
    b_per_w = total // NW
    n_gather = b_per_w // IDX_CHUNK
    w_per_seq = seq_len // b_per_w

    mesh = plsc.VectorSubcoreMesh(
        core_axis_name="c", subcore_axis_name="s", num_cores=NC, num_subcores=NS
    )

    @functools.partial(
        pl.kernel,
        out_type=jax.ShapeDtypeStruct((total, N_EMBD), jnp.float32),
        mesh=mesh,
        scratch_types=[
            pltpu.VMEM((b_per_w,), jnp.int32),
            pltpu.VMEM((b_per_w, N_EMBD), jnp.float32),
            pltpu.VMEM((b_per_w, N_EMBD), jnp.float32),
            pltpu.SemaphoreType.DMA,
            pltpu.SemaphoreType.DMA,
            pltpu.SemaphoreType.DMA,
            pltpu.SemaphoreType.DMA,
        ],
    )
    def sc_kernel(tok_hbm, w_hbm, cs_hbm, out_hbm,
                  idx_v, rows_v, cs_v, sem_t, sem_g0, sem_g1, sem_w):
        wid = lax.axis_index("s") * NC + lax.axis_index("c")
        base = wid * b_per_w
        bi = lax.div(wid, w_per_seq)
        pos_base = lax.rem(wid, w_per_seq) * b_per_w

        pltpu.sync_copy(tok_hbm.at[bi, pl.ds(pos_base, b_per_w)], idx_v)

        tab = pltpu.async_copy(cs_hbm.at[pl.ds(pos_base, b_per_w)], cs_v, sem_t)
        gsems = [sem_g0, sem_g1]
        gathers = [
            pltpu.async_copy(
                w_hbm.at[idx_v.at[pl.ds(g * IDX_CHUNK, IDX_CHUNK)]],
                rows_v.at[pl.ds(g * IDX_CHUNK, IDX_CHUNK)],
                gsems[g],
            )
            for g in range(n_gather)
        ]
        tab.wait()

        def rotate(t, carry):
            ts = [rows_v[t, pl.ds(j * L, L)] for j in range(N_EMBD // L)]
            cs = [cs_v[t, pl.ds(j * L, L)] for j in range(N_EMBD // L)]
            half = HALF // L
            for j in range(half):
                rows_v[t, pl.ds(j * L, L)] = ts[j] * cs[j] - ts[j + half] * cs[j + half]
                rows_v[t, pl.ds((j + half) * L, L)] = (
                    ts[j + half] * cs[j] + ts[j] * cs[j + half]
                )
            return carry

        writes = []
        for g in range(n_gather):
            gathers[g].wait()
            lax.fori_loop(g * IDX_CHUNK, (g + 1) * IDX_CHUNK, rotate, 0)
            writes.append(
                pltpu.async_copy(
                    rows_v.at[pl.ds(g * IDX_CHUNK, IDX_CHUNK)],
                    out_hbm.at[pl.ds(base + g * IDX_CHUNK, IDX_CHUNK)],
                    sem_w,
                )
            )
        for w in writes:
            w.wait()

    return sc_kernel


def kernel(token, W):
    batch, seq_len = token.shape
    cs = _rotary_cs_table(seq_len)
    sc = _make_sc_kernel(batch, seq_len)
    out = sc(token, W, cs)
    return out.reshape(batch, seq_len, N_EMBD)
